# initial kernel scaffold (unmeasured)
import jax
import jax.numpy as jnp
import numpy as np
from jax import lax
from jax.experimental import pallas as pl
from jax.experimental.pallas import tpu as pltpu

N_DEV = 4
SQ, D = 2048, 1024
H_PER, DH = 8, 128
RB = 512
SCALE = 0.08838834764831843

_sem_signal = getattr(pl, "semaphore_signal", None) or pltpu.semaphore_signal
_sem_wait = getattr(pl, "semaphore_wait", None) or pltpu.semaphore_wait
_DevIdType = getattr(pl, "DeviceIdType", None) or pltpu.DeviceIdType


def _rope_consts():
    inv = 1.0 / (10000.0 ** (np.arange(0, DH, 2) / DH))
    pos = np.arange(SQ)[:, None] * inv[None, :]
    cos = np.repeat(np.cos(pos), 2, axis=-1)
    sin = np.repeat(np.sin(pos), 2, axis=-1)
    cos_t = np.tile(cos, (1, H_PER))
    sin_t = np.tile(sin, (1, H_PER))
    P = np.zeros((D, D), np.float32)
    ev = np.arange(0, D, 2)
    P[ev + 1, ev] = -1.0
    P[ev, ev + 1] = 1.0
    return cos_t, sin_t, P


def _body(x_ref, wq_ref, wk_ref, wv_ref, wo_ref, cos_ref, sin_ref, p_ref,
          out_ref, q_ref, k_ref, v_ref, ctx_ref, comm_ref,
          send_sems, recv_sems):
    my = lax.axis_index("i")
    left = lax.rem(my + N_DEV - 1, N_DEV)
    right = lax.rem(my + 1, N_DEV)

    barrier_sem = pltpu.get_barrier_semaphore()
    for nbr in (left, right):
        _sem_signal(barrier_sem, inc=1, device_id=(nbr,),
                    device_id_type=_DevIdType.MESH)
    _sem_wait(barrier_sem, 2)

    xv = x_ref[...]
    cos = cos_ref[...].astype(jnp.float32)
    sin = sin_ref[...].astype(jnp.float32)
    p = p_ref[...]

    def project_rope(w_ref, dst_ref):
        t = jnp.dot(xv, w_ref[...], preferred_element_type=jnp.float32)
        tr = jnp.dot(t.astype(jnp.bfloat16), p,
                     preferred_element_type=jnp.float32)
        dst_ref[...] = (t * cos + tr * sin).astype(jnp.bfloat16)

    project_rope(wq_ref, q_ref)
    project_rope(wk_ref, k_ref)
    v_ref[...] = jnp.dot(xv, wv_ref[...],
                         preferred_element_type=jnp.float32).astype(jnp.bfloat16)

    for h in range(H_PER):
        hs = slice(h * DH, (h + 1) * DH)
        kh = k_ref[:, hs]
        vh = v_ref[:, hs]
        for rb in range(SQ // RB):
            rs = slice(rb * RB, (rb + 1) * RB)
            qh = q_ref[rs, hs]
            s = lax.dot_general(qh, kh, (((1,), (1,)), ((), ())),
                                preferred_element_type=jnp.float32) * SCALE
            m = jnp.max(s, axis=-1, keepdims=True)
            e = jnp.exp(s - m)
            w = (e / jnp.sum(e, axis=-1, keepdims=True)).astype(jnp.bfloat16)
            ctx_ref[rs, hs] = jnp.dot(
                w, vh, preferred_element_type=jnp.float32).astype(jnp.bfloat16)

    acc = jnp.dot(ctx_ref[...], wo_ref[...],
                  preferred_element_type=jnp.float32)
    out_ref[...] = acc
    comm_ref[0] = acc.astype(jnp.bfloat16)

    for hop in range(N_DEV - 1):
        send_slot = hop % 2
        recv_slot = (hop + 1) % 2
        rdma = pltpu.make_async_remote_copy(
            src_ref=comm_ref.at[send_slot],
            dst_ref=comm_ref.at[recv_slot],
            send_sem=send_sems.at[send_slot],
            recv_sem=recv_sems.at[recv_slot],
            device_id=(right,),
            device_id_type=_DevIdType.MESH,
        )
        rdma.start()
        rdma.wait()
        out_ref[...] += comm_ref[recv_slot].astype(jnp.float32)


def kernel(x, Wq, Wk, Wv, Wo):
    cos_np, sin_np, p_np = _rope_consts()
    args = (
        x.reshape(SQ, D).astype(jnp.bfloat16),
        Wq.astype(jnp.bfloat16),
        Wk.astype(jnp.bfloat16),
        Wv.astype(jnp.bfloat16),
        Wo.astype(jnp.bfloat16),
        jnp.asarray(cos_np, jnp.bfloat16),
        jnp.asarray(sin_np, jnp.bfloat16),
        jnp.asarray(p_np, jnp.bfloat16),
    )
    out = pl.pallas_call(
        _body,
        out_shape=jax.ShapeDtypeStruct((SQ, D), jnp.float32),
        in_specs=[pl.BlockSpec(memory_space=pltpu.VMEM)] * len(args),
        out_specs=pl.BlockSpec(memory_space=pltpu.VMEM),
        scratch_shapes=[
            pltpu.VMEM((SQ, D), jnp.bfloat16),
            pltpu.VMEM((SQ, D), jnp.bfloat16),
            pltpu.VMEM((SQ, D), jnp.bfloat16),
            pltpu.VMEM((SQ, D), jnp.bfloat16),
            pltpu.VMEM((2, SQ, D), jnp.bfloat16),
            pltpu.SemaphoreType.DMA((2,)),
            pltpu.SemaphoreType.DMA((2,)),
        ],
        compiler_params=pltpu.CompilerParams(collective_id=0),
    )(*args)
    return out.reshape(1, SQ, D)


# baseline (device time: 297758 ns/iter reference)
import jax
import jax.numpy as jnp
import numpy as np
from jax import lax
from jax.experimental import pallas as pl
from jax.experimental.pallas import tpu as pltpu

N_DEV = 4
SQ, D = 2048, 1024
H_PER, DH = 8, 128
RB = 512
SCALE = 0.08838834764831843

_sem_signal = getattr(pl, "semaphore_signal", None) or pltpu.semaphore_signal
_sem_wait = getattr(pl, "semaphore_wait", None) or pltpu.semaphore_wait
_DevIdType = getattr(pl, "DeviceIdType", None) or pltpu.DeviceIdType


def _rope_consts():
    inv = 1.0 / (10000.0 ** (np.arange(0, DH, 2) / DH))
    pos = np.arange(SQ)[:, None] * inv[None, :]
    cos = np.repeat(np.cos(pos), 2, axis=-1)
    sin = np.repeat(np.sin(pos), 2, axis=-1)
    P = np.zeros((DH, DH), np.float32)
    ev = np.arange(0, DH, 2)
    P[ev + 1, ev] = -1.0
    P[ev, ev + 1] = 1.0
    return cos, sin, P


def _body(x_ref, wq_ref, wk_ref, wv_ref, wo_ref, cos_ref, sin_ref, p_ref,
          out_ref, q_ref, k_ref, v_ref, comm_ref, send_sems, recv_sems):
    my = lax.axis_index("i")
    left = lax.rem(my + N_DEV - 1, N_DEV)
    right = lax.rem(my + 1, N_DEV)

    barrier_sem = pltpu.get_barrier_semaphore()
    for nbr in (left, right):
        _sem_signal(barrier_sem, inc=1, device_id=(nbr,),
                    device_id_type=_DevIdType.MESH)
    _sem_wait(barrier_sem, 2)

    xv = x_ref[...]
    q_ref[...] = jnp.dot(
        xv, wq_ref[...], preferred_element_type=jnp.float32).astype(jnp.bfloat16)
    k_ref[...] = jnp.dot(
        xv, wk_ref[...], preferred_element_type=jnp.float32).astype(jnp.bfloat16)
    v_ref[...] = jnp.dot(
        xv, wv_ref[...], preferred_element_type=jnp.float32).astype(jnp.bfloat16)

    p = p_ref[...]
    cos = cos_ref[...]
    sin = sin_ref[...]

    def rope(t, rows):
        tr = jnp.dot(t, p, preferred_element_type=jnp.float32).astype(jnp.bfloat16)
        return t * cos[rows] + tr * sin[rows]

    for h in range(H_PER):
        hs = slice(h * DH, (h + 1) * DH)
        kh = rope(k_ref[:, hs], slice(None))
        vh = v_ref[:, hs]
        woh = wo_ref[hs, :]
        for rb in range(SQ // RB):
            rs = slice(rb * RB, (rb + 1) * RB)
            qh = rope(q_ref[rs, hs], rs)
            s = lax.dot_general(qh, kh, (((1,), (1,)), ((), ())),
                                preferred_element_type=jnp.float32) * SCALE
            m = jnp.max(s, axis=-1, keepdims=True)
            e = jnp.exp(s - m)
            w = (e / jnp.sum(e, axis=-1, keepdims=True)).astype(jnp.bfloat16)
            ctx = jnp.dot(
                w, vh, preferred_element_type=jnp.float32).astype(jnp.bfloat16)
            part = jnp.dot(ctx, woh, preferred_element_type=jnp.float32)
            if h == 0:
                out_ref[rs, :] = part
            else:
                out_ref[rs, :] += part

    comm_ref[0] = out_ref[...].astype(jnp.bfloat16)

    for hop in range(N_DEV - 1):
        send_slot = hop % 2
        recv_slot = (hop + 1) % 2
        rdma = pltpu.make_async_remote_copy(
            src_ref=comm_ref.at[send_slot],
            dst_ref=comm_ref.at[recv_slot],
            send_sem=send_sems.at[send_slot],
            recv_sem=recv_sems.at[recv_slot],
            device_id=(right,),
            device_id_type=_DevIdType.MESH,
        )
        rdma.start()
        rdma.wait()
        out_ref[...] += comm_ref[recv_slot].astype(jnp.float32)


def kernel(x, Wq, Wk, Wv, Wo):
    cos_np, sin_np, p_np = _rope_consts()
    args = (
        x.reshape(SQ, D).astype(jnp.bfloat16),
        Wq.astype(jnp.bfloat16),
        Wk.astype(jnp.bfloat16),
        Wv.astype(jnp.bfloat16),
        Wo.astype(jnp.bfloat16),
        jnp.asarray(cos_np, jnp.bfloat16),
        jnp.asarray(sin_np, jnp.bfloat16),
        jnp.asarray(p_np, jnp.bfloat16),
    )
    out = pl.pallas_call(
        _body,
        out_shape=jax.ShapeDtypeStruct((SQ, D), jnp.float32),
        in_specs=[pl.BlockSpec(memory_space=pltpu.VMEM)] * len(args),
        out_specs=pl.BlockSpec(memory_space=pltpu.VMEM),
        scratch_shapes=[
            pltpu.VMEM((SQ, D), jnp.bfloat16),
            pltpu.VMEM((SQ, D), jnp.bfloat16),
            pltpu.VMEM((SQ, D), jnp.bfloat16),
            pltpu.VMEM((2, SQ, D), jnp.bfloat16),
            pltpu.SemaphoreType.DMA((2,)),
            pltpu.SemaphoreType.DMA((2,)),
        ],
        compiler_params=pltpu.CompilerParams(
            collective_id=0, vmem_limit_bytes=100 * 1024 * 1024),
    )(*args)
    return out.reshape(1, SQ, D)


# device time: 181261 ns/iter; 1.6427x vs baseline; 1.6427x over previous
import jax
import jax.numpy as jnp
import numpy as np
from jax import lax
from jax.experimental import pallas as pl
from jax.experimental.pallas import tpu as pltpu

N_DEV = 4
SQ, D = 2048, 1024
H_PER, DH = 8, 128
CH = SQ // N_DEV
SCALE = 0.08838834764831843

_sem_signal = getattr(pl, "semaphore_signal", None) or pltpu.semaphore_signal
_sem_wait = getattr(pl, "semaphore_wait", None) or pltpu.semaphore_wait
_DevIdType = getattr(pl, "DeviceIdType", None) or pltpu.DeviceIdType


def _rope_consts():
    inv = 1.0 / (10000.0 ** (np.arange(0, DH, 2) / DH))
    pos = np.arange(SQ)[:, None] * inv[None, :]
    cos = np.repeat(np.cos(pos), 2, axis=-1)
    sin = np.repeat(np.sin(pos), 2, axis=-1)
    P = np.zeros((DH, DH), np.float32)
    ev = np.arange(0, DH, 2)
    P[ev + 1, ev] = -1.0
    P[ev, ev + 1] = 1.0
    return cos, sin, P


def _body(x_ref, wq_ref, wk_ref, wv_ref, wo_ref, cos_ref, sin_ref, p_ref,
          out_ref, q_ref, k_ref, v_ref, rs_send, rs_recv, ag_buf,
          send_sems, recv_sems):
    my = lax.axis_index("i")
    left = lax.rem(my + N_DEV - 1, N_DEV)
    right = lax.rem(my + 1, N_DEV)

    barrier_sem = pltpu.get_barrier_semaphore()
    for nbr in (left, right):
        _sem_signal(barrier_sem, inc=1, device_id=(nbr,),
                    device_id_type=_DevIdType.MESH)
    _sem_wait(barrier_sem, 2)

    xv = x_ref[...]
    q_ref[...] = jnp.dot(
        xv, wq_ref[...], preferred_element_type=jnp.float32).astype(jnp.bfloat16)
    k_ref[...] = jnp.dot(
        xv, wk_ref[...], preferred_element_type=jnp.float32).astype(jnp.bfloat16)
    v_ref[...] = jnp.dot(
        xv, wv_ref[...], preferred_element_type=jnp.float32).astype(jnp.bfloat16)

    p = p_ref[...]
    cos = cos_ref[...]
    sin = sin_ref[...]

    def rope(t, c, s):
        tr = jnp.dot(t, p, preferred_element_type=jnp.float32).astype(jnp.bfloat16)
        return t * c + tr * s

    for h in range(H_PER):
        hs = slice(h * DH, (h + 1) * DH)
        k_ref[:, hs] = rope(k_ref[:, hs], cos, sin)

    def chunk_rows(t):
        return pl.ds(lax.rem(my - t + N_DEV, N_DEV) * CH, CH)

    def compute_chunk(t):
        rows = chunk_rows(t)
        cos_r = cos_ref[rows, :]
        sin_r = sin_ref[rows, :]
        acc = None
        for h in range(H_PER):
            hs = slice(h * DH, (h + 1) * DH)
            qh = rope(q_ref[rows, hs], cos_r, sin_r)
            s = lax.dot_general(qh, k_ref[:, hs], (((1,), (1,)), ((), ())),
                                preferred_element_type=jnp.float32) * SCALE
            m = jnp.max(s, axis=-1, keepdims=True)
            e = jnp.exp(s - m)
            w = (e / jnp.sum(e, axis=-1, keepdims=True)).astype(jnp.bfloat16)
            ctx = jnp.dot(
                w, v_ref[:, hs], preferred_element_type=jnp.float32
            ).astype(jnp.bfloat16)
            part = jnp.dot(ctx, wo_ref[hs, :], preferred_element_type=jnp.float32)
            acc = part if acc is None else acc + part
        out_ref[rows, :] = acc

    def hop(src, dst, sem_idx):
        return pltpu.make_async_remote_copy(
            src_ref=src, dst_ref=dst,
            send_sem=send_sems.at[sem_idx], recv_sem=recv_sems.at[sem_idx],
            device_id=(right,), device_id_type=_DevIdType.MESH,
        )

    compute_chunk(0)
    rs_send[0] = out_ref[chunk_rows(0), :].astype(jnp.bfloat16)
    h0 = hop(rs_send.at[0], rs_recv.at[0], 0)
    h0.start()
    hops = [h0]
    for j in (1, 2):
        compute_chunk(j)
        hops[j - 1].wait()
        rs_send[j] = (rs_recv[j - 1].astype(jnp.float32)
                      + out_ref[chunk_rows(j), :]).astype(jnp.bfloat16)
        hj = hop(rs_send.at[j], rs_recv.at[j], j)
        hj.start()
        hops.append(hj)
    compute_chunk(3)
    hops[2].wait()
    final = rs_recv[2].astype(jnp.float32) + out_ref[chunk_rows(3), :]
    out_ref[chunk_rows(3), :] = final
    ag_buf[0] = final.astype(jnp.bfloat16)

    for a in range(N_DEV - 1):
        ha = hop(ag_buf.at[a], ag_buf.at[a + 1], 3 + a)
        ha.start()
        ha.wait()
        out_ref[chunk_rows(a), :] = ag_buf[a + 1].astype(jnp.float32)


def kernel(x, Wq, Wk, Wv, Wo):
    cos_np, sin_np, p_np = _rope_consts()
    args = (
        x.reshape(SQ, D).astype(jnp.bfloat16),
        Wq.astype(jnp.bfloat16),
        Wk.astype(jnp.bfloat16),
        Wv.astype(jnp.bfloat16),
        Wo.astype(jnp.bfloat16),
        jnp.asarray(cos_np, jnp.bfloat16),
        jnp.asarray(sin_np, jnp.bfloat16),
        jnp.asarray(p_np, jnp.bfloat16),
    )
    out = pl.pallas_call(
        _body,
        out_shape=jax.ShapeDtypeStruct((SQ, D), jnp.float32),
        in_specs=[pl.BlockSpec(memory_space=pltpu.VMEM)] * len(args),
        out_specs=pl.BlockSpec(memory_space=pltpu.VMEM),
        scratch_shapes=[
            pltpu.VMEM((SQ, D), jnp.bfloat16),
            pltpu.VMEM((SQ, D), jnp.bfloat16),
            pltpu.VMEM((SQ, D), jnp.bfloat16),
            pltpu.VMEM((3, CH, D), jnp.bfloat16),
            pltpu.VMEM((3, CH, D), jnp.bfloat16),
            pltpu.VMEM((4, CH, D), jnp.bfloat16),
            pltpu.SemaphoreType.DMA((6,)),
            pltpu.SemaphoreType.DMA((6,)),
        ],
        compiler_params=pltpu.CompilerParams(
            collective_id=0, vmem_limit_bytes=100 * 1024 * 1024),
    )(*args)
    return out.reshape(1, SQ, D)


# device time: 136530 ns/iter; 2.1809x vs baseline; 1.3276x over previous
import jax
import jax.numpy as jnp
import numpy as np
from jax import lax
from jax.experimental import pallas as pl
from jax.experimental.pallas import tpu as pltpu

N_DEV = 4
SQ, D = 2048, 1024
H_PER, DH = 8, 128
CH = SQ // N_DEV
HCH = CH // 2
SCALE = 0.08838834764831843

_sem_signal = getattr(pl, "semaphore_signal", None) or pltpu.semaphore_signal
_sem_wait = getattr(pl, "semaphore_wait", None) or pltpu.semaphore_wait
_DevIdType = getattr(pl, "DeviceIdType", None) or pltpu.DeviceIdType


def _rope_consts():
    inv = 1.0 / (10000.0 ** (np.arange(0, DH, 2) / DH))
    pos = np.arange(SQ)[:, None] * inv[None, :]
    cos = np.repeat(np.cos(pos), 2, axis=-1)
    sin = np.repeat(np.sin(pos), 2, axis=-1)
    P = np.zeros((DH, DH), np.float32)
    ev = np.arange(0, DH, 2)
    P[ev + 1, ev] = -1.0
    P[ev, ev + 1] = 1.0
    return cos, sin, P


def _body(x_ref, wq_ref, wk_ref, wv_ref, wo_ref, cos_ref, sin_ref, p_ref,
          out_ref, q_ref, k_ref, v_ref, rs_send, rs_recv, ag_r, ag_l,
          send_sems, recv_sems):
    my = lax.axis_index("i")
    left = lax.rem(my + N_DEV - 1, N_DEV)
    right = lax.rem(my + 1, N_DEV)

    barrier_sem = pltpu.get_barrier_semaphore()
    for nbr in (left, right):
        _sem_signal(barrier_sem, inc=1, device_id=(nbr,),
                    device_id_type=_DevIdType.MESH)
    _sem_wait(barrier_sem, 2)

    xv = x_ref[...]
    q_ref[...] = jnp.dot(
        xv, wq_ref[...], preferred_element_type=jnp.float32).astype(jnp.bfloat16)
    k_ref[...] = jnp.dot(
        xv, wk_ref[...], preferred_element_type=jnp.float32).astype(jnp.bfloat16)
    v_ref[...] = jnp.dot(
        xv, wv_ref[...], preferred_element_type=jnp.float32).astype(jnp.bfloat16)

    p = p_ref[...]
    cos = cos_ref[...]
    sin = sin_ref[...]

    def rope(t, c, s):
        tr = jnp.dot(t, p, preferred_element_type=jnp.float32).astype(jnp.bfloat16)
        return t * c + tr * s

    for h in range(H_PER):
        hs = slice(h * DH, (h + 1) * DH)
        k_ref[:, hs] = rope(k_ref[:, hs], cos, sin)

    def chunk_rows(t):
        return pl.ds(lax.rem(my - t + N_DEV, N_DEV) * CH, CH)

    def compute_chunk(t):
        rows = chunk_rows(t)
        cos_r = cos_ref[rows, :]
        sin_r = sin_ref[rows, :]
        acc = None
        for h in range(H_PER):
            hs = slice(h * DH, (h + 1) * DH)
            qh = rope(q_ref[rows, hs], cos_r, sin_r)
            s = lax.dot_general(qh, k_ref[:, hs], (((1,), (1,)), ((), ())),
                                preferred_element_type=jnp.float32) * SCALE
            e = jnp.exp(s)
            denom = jnp.sum(e, axis=-1, keepdims=True)
            ctx = jnp.dot(
                e.astype(jnp.bfloat16), v_ref[:, hs],
                preferred_element_type=jnp.float32)
            ctx = (ctx * (1.0 / denom)).astype(jnp.bfloat16)
            part = jnp.dot(ctx, wo_ref[hs, :], preferred_element_type=jnp.float32)
            acc = part if acc is None else acc + part
        out_ref[rows, :] = acc

    def hop(src, dst, sem_idx, dst_dev=None):
        return pltpu.make_async_remote_copy(
            src_ref=src, dst_ref=dst,
            send_sem=send_sems.at[sem_idx], recv_sem=recv_sems.at[sem_idx],
            device_id=(right if dst_dev is None else dst_dev,),
            device_id_type=_DevIdType.MESH,
        )

    compute_chunk(0)
    rs_send[0] = out_ref[chunk_rows(0), :].astype(jnp.bfloat16)
    h0 = hop(rs_send.at[0], rs_recv.at[0], 0)
    h0.start()
    hops = [h0]
    for j in (1, 2):
        compute_chunk(j)
        hops[j - 1].wait()
        rs_send[j] = (rs_recv[j - 1].astype(jnp.float32)
                      + out_ref[chunk_rows(j), :]).astype(jnp.bfloat16)
        hj = hop(rs_send.at[j], rs_recv.at[j], j)
        hj.start()
        hops.append(hj)
    compute_chunk(3)
    hops[2].wait()
    final = rs_recv[2].astype(jnp.float32) + out_ref[chunk_rows(3), :]
    out_ref[chunk_rows(3), :] = final
    ag_r[0] = final[:HCH, :].astype(jnp.bfloat16)
    ag_l[0] = final[HCH:, :].astype(jnp.bfloat16)

    for a in range(N_DEV - 1):
        hr = hop(ag_r.at[a], ag_r.at[a + 1], 3 + a)
        hl = hop(ag_l.at[a], ag_l.at[a + 1], 6 + a, dst_dev=left)
        hr.start()
        hl.start()
        hr.wait()
        hl.wait()
        r_start = lax.rem(my - a + N_DEV, N_DEV) * CH
        l_start = lax.rem(my + 2 + a, N_DEV) * CH + HCH
        out_ref[pl.ds(r_start, HCH), :] = ag_r[a + 1].astype(jnp.float32)
        out_ref[pl.ds(l_start, HCH), :] = ag_l[a + 1].astype(jnp.float32)


def kernel(x, Wq, Wk, Wv, Wo):
    cos_np, sin_np, p_np = _rope_consts()
    args = (
        x.reshape(SQ, D).astype(jnp.bfloat16),
        Wq.astype(jnp.bfloat16),
        Wk.astype(jnp.bfloat16),
        Wv.astype(jnp.bfloat16),
        Wo.astype(jnp.bfloat16),
        jnp.asarray(cos_np, jnp.bfloat16),
        jnp.asarray(sin_np, jnp.bfloat16),
        jnp.asarray(p_np, jnp.bfloat16),
    )
    out = pl.pallas_call(
        _body,
        out_shape=jax.ShapeDtypeStruct((SQ, D), jnp.float32),
        in_specs=[pl.BlockSpec(memory_space=pltpu.VMEM)] * len(args),
        out_specs=pl.BlockSpec(memory_space=pltpu.VMEM),
        scratch_shapes=[
            pltpu.VMEM((SQ, D), jnp.bfloat16),
            pltpu.VMEM((SQ, D), jnp.bfloat16),
            pltpu.VMEM((SQ, D), jnp.bfloat16),
            pltpu.VMEM((3, CH, D), jnp.bfloat16),
            pltpu.VMEM((3, CH, D), jnp.bfloat16),
            pltpu.VMEM((4, HCH, D), jnp.bfloat16),
            pltpu.VMEM((4, HCH, D), jnp.bfloat16),
            pltpu.SemaphoreType.DMA((9,)),
            pltpu.SemaphoreType.DMA((9,)),
        ],
        compiler_params=pltpu.CompilerParams(
            collective_id=0, vmem_limit_bytes=100 * 1024 * 1024),
    )(*args)
    return out.reshape(1, SQ, D)
